# baseline (speedup 1.0000x reference)
import dataclasses

import jax
import jax.numpy as jnp
from jax import lax
from jax.experimental import pallas as pl
from jax.experimental.pallas import tpu as pltpu
from jax.experimental.pallas import tpu_sc as plsc

N = 4_000_000
NUM_IMGS = 5000
L = 16
NW = 32
C = 6144
GROUPS = N // L
G_BASE = GROUPS // NW
G_REM = GROUPS - G_BASE * NW
MAX_COUNT = (G_BASE + 1) * L
CHUNKS = (MAX_COUNT + C - 1) // C + 1


def _body(imgidx_hbm, ptidx_hbm, consts_hbm, ext_hbm,
          xs_hbm, ys_hbm, zs_hbm, ou_hbm, ov_hbm,
          ext_v, consts_v,
          idx_img_v0, idx_pt_v0, xg_v0, yg_v0, zg_v0,
          idx_img_v1, idx_pt_v1, xg_v1, yg_v1, zg_v1,
          ou_v, ov_v, sem0, sem1):
    cid = lax.axis_index("c")
    sid = lax.axis_index("s")
    wid = sid * 2 + cid

    pltpu.sync_copy(ext_hbm, ext_v)
    pltpu.sync_copy(consts_hbm, consts_v)
    fvec = consts_v[pl.ds(0, L)]
    ppx = consts_v[pl.ds(L, L)]
    ppy = consts_v[pl.ds(2 * L, L)]

    count_g = G_BASE + jnp.where(wid < G_REM, 1, 0)
    start = (wid * G_BASE + jnp.minimum(wid, G_REM)) * L
    end = start + count_g * L

    iota = lax.iota(jnp.int32, L)
    sets = ((idx_img_v0, idx_pt_v0, xg_v0, yg_v0, zg_v0, sem0),
            (idx_img_v1, idx_pt_v1, xg_v1, yg_v1, zg_v1, sem1))

    def chunk_st(i):
        return pl.multiple_of(jnp.minimum(start + i * C, end - C), L)

    def load_chunk(i, s):
        idx_img, idx_pt, xg, yg, zg, sem = s
        st = chunk_st(i)
        pltpu.sync_copy(imgidx_hbm.at[pl.ds(st, C)], idx_img)
        pltpu.sync_copy(ptidx_hbm.at[pl.ds(st, C)], idx_pt)
        pltpu.async_copy(xs_hbm.at[idx_pt], xg, sem)
        pltpu.async_copy(ys_hbm.at[idx_pt], yg, sem)
        pltpu.async_copy(zs_hbm.at[idx_pt], zg, sem)

    def wait_chunk(s):
        idx_img, idx_pt, xg, yg, zg, sem = s
        pltpu.make_async_copy(xs_hbm.at[idx_pt], xg, sem).wait()
        pltpu.make_async_copy(ys_hbm.at[idx_pt], yg, sem).wait()
        pltpu.make_async_copy(zs_hbm.at[idx_pt], zg, sem).wait()

    def compute_chunk(i, s):
        idx_img, idx_pt, xg_v, yg_v, zg_v, sem = s
        st = chunk_st(i)

        @pl.loop(0, C // L)
        def _group(g):
            b = pl.multiple_of(g * L, L)
            row = iota + b
            e = idx_img[pl.ds(b, L)] * 7
            qw = plsc.load_gather(ext_v, [e])
            qx = plsc.load_gather(ext_v, [e + 1])
            qy = plsc.load_gather(ext_v, [e + 2])
            qz = plsc.load_gather(ext_v, [e + 3])
            tx = plsc.load_gather(ext_v, [e + 4])
            ty = plsc.load_gather(ext_v, [e + 5])
            tz = plsc.load_gather(ext_v, [e + 6])
            px = xg_v[pl.ds(b, L)]
            py = yg_v[pl.ds(b, L)]
            pz = zg_v[pl.ds(b, L)]
            ax = qy * pz - qz * py
            ay = qz * px - qx * pz
            az = qx * py - qy * px
            bx = ax + qw * px
            by = ay + qw * py
            bz = az + qw * pz
            cx = qy * bz - qz * by
            cy = qz * bx - qx * bz
            cz = qx * by - qy * bx
            pcx = px + cx + cx + tx
            pcy = py + cy + cy + ty
            pcz = pz + cz + cz + tz
            d = fvec / pcz
            plsc.store_scatter(ou_v, [row], pcx * d + ppx)
            plsc.store_scatter(ov_v, [row], pcy * d + ppy)

        pltpu.sync_copy(ou_v, ou_hbm.at[pl.ds(st, C)])
        pltpu.sync_copy(ov_v, ov_hbm.at[pl.ds(st, C)])

    load_chunk(0, sets[0])

    @pl.loop(0, CHUNKS // 2)
    def _k(k):
        i0 = k + k
        for p in range(2):
            i = i0 + p
            nxt = i + 1

            @pl.when(nxt < CHUNKS)
            def _prefetch():
                load_chunk(nxt, sets[(p + 1) % 2])

            wait_chunk(sets[p])
            compute_chunk(i, sets[p])


def kernel(points_2d, image_indices, camera_indices, point_indices,
           camera_pps, extrinsics, intrinsics, points_3d):
    del camera_indices
    consts = jnp.concatenate([
        jnp.broadcast_to(intrinsics[0, 0], (L,)),
        jnp.broadcast_to(camera_pps[0, 0], (L,)),
        jnp.broadcast_to(camera_pps[0, 1], (L,)),
    ]).astype(jnp.float32)
    ext_flat = extrinsics.reshape(-1)
    mesh = plsc.VectorSubcoreMesh(core_axis_name="c", subcore_axis_name="s")
    cp = pltpu.CompilerParams()
    if "needs_layout_passes" in pltpu.CompilerParams.__dataclass_fields__:
        cp = dataclasses.replace(cp, needs_layout_passes=False)
    cp = dataclasses.replace(cp, use_tc_tiling_on_sc=False)
    run = pl.kernel(
        _body,
        compiler_params=cp,
        out_type=(jax.ShapeDtypeStruct((N,), jnp.float32),
                  jax.ShapeDtypeStruct((N,), jnp.float32)),
        mesh=mesh,
        scratch_types=[
            pltpu.VMEM((NUM_IMGS * 7,), jnp.float32),
            pltpu.VMEM((3 * L,), jnp.float32),
            pltpu.VMEM((C,), jnp.int32),
            pltpu.VMEM((C,), jnp.int32),
            pltpu.VMEM((C,), jnp.float32),
            pltpu.VMEM((C,), jnp.float32),
            pltpu.VMEM((C,), jnp.float32),
            pltpu.VMEM((C,), jnp.int32),
            pltpu.VMEM((C,), jnp.int32),
            pltpu.VMEM((C,), jnp.float32),
            pltpu.VMEM((C,), jnp.float32),
            pltpu.VMEM((C,), jnp.float32),
            pltpu.VMEM((C,), jnp.float32),
            pltpu.VMEM((C,), jnp.float32),
            pltpu.SemaphoreType.DMA,
            pltpu.SemaphoreType.DMA,
        ],
    )
    ou, ov = run(image_indices.astype(jnp.int32),
                 point_indices.astype(jnp.int32), consts, ext_flat,
                 points_3d[:, 0], points_3d[:, 1], points_3d[:, 2])
    return jnp.stack([ou, ov], axis=1) - points_2d


# speedup vs baseline: 1.0226x; 1.0226x over previous
import dataclasses

import jax
import jax.numpy as jnp
from jax import lax
from jax.experimental import pallas as pl
from jax.experimental.pallas import tpu as pltpu
from jax.experimental.pallas import tpu_sc as plsc

N = 4_000_000
NUM_IMGS = 5000
L = 16
NW = 32
C = 4096
GROUPS = N // L
G_BASE = GROUPS // NW
G_REM = GROUPS - G_BASE * NW
MAX_COUNT = (G_BASE + 1) * L
CHUNKS = (MAX_COUNT + C - 1) // C + 1


def _body(imgidx_hbm, ptidx_hbm, consts_hbm, ext_hbm,
          xs_hbm, ys_hbm, zs_hbm, ou_hbm, ov_hbm,
          ext_v, consts_v,
          idx_img_v0, idx_pt_v0, xg_v0, yg_v0, zg_v0,
          idx_img_v1, idx_pt_v1, xg_v1, yg_v1, zg_v1,
          ou_v, ov_v, sem0, sem1):
    cid = lax.axis_index("c")
    sid = lax.axis_index("s")
    wid = sid * 2 + cid

    pltpu.sync_copy(ext_hbm, ext_v)
    pltpu.sync_copy(consts_hbm, consts_v)
    fvec = consts_v[pl.ds(0, L)]
    ppx = consts_v[pl.ds(L, L)]
    ppy = consts_v[pl.ds(2 * L, L)]

    count_g = G_BASE + jnp.where(wid < G_REM, 1, 0)
    start = (wid * G_BASE + jnp.minimum(wid, G_REM)) * L
    end = start + count_g * L

    iota = lax.iota(jnp.int32, L)
    sets = ((idx_img_v0, idx_pt_v0, xg_v0, yg_v0, zg_v0, sem0),
            (idx_img_v1, idx_pt_v1, xg_v1, yg_v1, zg_v1, sem1))

    def chunk_st(i):
        return pl.multiple_of(jnp.minimum(start + i * C, end - C), L)

    def load_chunk(i, s):
        idx_img, idx_pt, xg, yg, zg, sem = s
        st = chunk_st(i)
        pltpu.sync_copy(imgidx_hbm.at[pl.ds(st, C)], idx_img)
        pltpu.sync_copy(ptidx_hbm.at[pl.ds(st, C)], idx_pt)
        pltpu.async_copy(xs_hbm.at[idx_pt], xg, sem)
        pltpu.async_copy(ys_hbm.at[idx_pt], yg, sem)
        pltpu.async_copy(zs_hbm.at[idx_pt], zg, sem)

    def wait_chunk(s):
        idx_img, idx_pt, xg, yg, zg, sem = s
        pltpu.make_async_copy(xs_hbm.at[idx_pt], xg, sem).wait()
        pltpu.make_async_copy(ys_hbm.at[idx_pt], yg, sem).wait()
        pltpu.make_async_copy(zs_hbm.at[idx_pt], zg, sem).wait()

    def compute_chunk(i, s):
        idx_img, idx_pt, xg_v, yg_v, zg_v, sem = s
        st = chunk_st(i)

        @pl.loop(0, C // L)
        def _group(g):
            b = pl.multiple_of(g * L, L)
            row = iota + b
            e = idx_img[pl.ds(b, L)] * 7
            qw = plsc.load_gather(ext_v, [e])
            qx = plsc.load_gather(ext_v, [e + 1])
            qy = plsc.load_gather(ext_v, [e + 2])
            qz = plsc.load_gather(ext_v, [e + 3])
            tx = plsc.load_gather(ext_v, [e + 4])
            ty = plsc.load_gather(ext_v, [e + 5])
            tz = plsc.load_gather(ext_v, [e + 6])
            px = xg_v[pl.ds(b, L)]
            py = yg_v[pl.ds(b, L)]
            pz = zg_v[pl.ds(b, L)]
            ax = qy * pz - qz * py
            ay = qz * px - qx * pz
            az = qx * py - qy * px
            bx = ax + qw * px
            by = ay + qw * py
            bz = az + qw * pz
            cx = qy * bz - qz * by
            cy = qz * bx - qx * bz
            cz = qx * by - qy * bx
            pcx = px + cx + cx + tx
            pcy = py + cy + cy + ty
            pcz = pz + cz + cz + tz
            d = fvec / pcz
            plsc.store_scatter(ou_v, [row], pcx * d + ppx)
            plsc.store_scatter(ov_v, [row], pcy * d + ppy)

        pltpu.sync_copy(ou_v, ou_hbm.at[pl.ds(st, C)])
        pltpu.sync_copy(ov_v, ov_hbm.at[pl.ds(st, C)])

    load_chunk(0, sets[0])

    @pl.loop(0, CHUNKS // 2)
    def _k(k):
        i0 = k + k
        for p in range(2):
            i = i0 + p
            nxt = i + 1

            @pl.when(nxt < CHUNKS)
            def _prefetch():
                load_chunk(nxt, sets[(p + 1) % 2])

            wait_chunk(sets[p])
            compute_chunk(i, sets[p])


def kernel(points_2d, image_indices, camera_indices, point_indices,
           camera_pps, extrinsics, intrinsics, points_3d):
    del camera_indices
    consts = jnp.concatenate([
        jnp.broadcast_to(intrinsics[0, 0], (L,)),
        jnp.broadcast_to(camera_pps[0, 0], (L,)),
        jnp.broadcast_to(camera_pps[0, 1], (L,)),
    ]).astype(jnp.float32)
    ext_flat = extrinsics.reshape(-1)
    mesh = plsc.VectorSubcoreMesh(core_axis_name="c", subcore_axis_name="s")
    cp = pltpu.CompilerParams()
    if "needs_layout_passes" in pltpu.CompilerParams.__dataclass_fields__:
        cp = dataclasses.replace(cp, needs_layout_passes=False)
    cp = dataclasses.replace(cp, use_tc_tiling_on_sc=False)
    run = pl.kernel(
        _body,
        compiler_params=cp,
        out_type=(jax.ShapeDtypeStruct((N,), jnp.float32),
                  jax.ShapeDtypeStruct((N,), jnp.float32)),
        mesh=mesh,
        scratch_types=[
            pltpu.VMEM((NUM_IMGS * 7,), jnp.float32),
            pltpu.VMEM((3 * L,), jnp.float32),
            pltpu.VMEM((C,), jnp.int32),
            pltpu.VMEM((C,), jnp.int32),
            pltpu.VMEM((C,), jnp.float32),
            pltpu.VMEM((C,), jnp.float32),
            pltpu.VMEM((C,), jnp.float32),
            pltpu.VMEM((C,), jnp.int32),
            pltpu.VMEM((C,), jnp.int32),
            pltpu.VMEM((C,), jnp.float32),
            pltpu.VMEM((C,), jnp.float32),
            pltpu.VMEM((C,), jnp.float32),
            pltpu.VMEM((C,), jnp.float32),
            pltpu.VMEM((C,), jnp.float32),
            pltpu.SemaphoreType.DMA,
            pltpu.SemaphoreType.DMA,
        ],
    )
    ou, ov = run(image_indices.astype(jnp.int32),
                 point_indices.astype(jnp.int32), consts, ext_flat,
                 points_3d[:, 0], points_3d[:, 1], points_3d[:, 2])
    return jnp.stack([ou, ov], axis=1) - points_2d
